# Initial kernel scaffold; baseline (speedup 1.0000x reference)
import jax
import jax.numpy as jnp
from jax.experimental import pallas as pl


def kernel(x, pc, fea_w, fea_b, fea_bn_g, fea_bn_b, fea_bn_m, fea_bn_v, xyz_w, xyz_b, xyz_bn_g, xyz_bn_b, xyz_bn_m, xyz_bn_v, a1_w, a1_b, a1_bn_g, a1_bn_b, a1_bn_m, a1_bn_v, a2_w, a2_b, a2_bn_g, a2_bn_b, a2_bn_m, a2_bn_v, hk_w, hk_b, hk_bn_g, hk_bn_b, hk_bn_m, hk_bn_v, c2_w, c2_b, c2_bn_g, c2_bn_b, c2_bn_m, c2_bn_v, uc_bn_g, uc_bn_b, uc_bn_m, uc_bn_v, fc1_w, fc1_b, fb1_bn_g, fb1_bn_b, fb1_bn_m, fb1_bn_v, fc2_w, fc2_b, fb2_bn_g, fb2_bn_b, fb2_bn_m, fb2_bn_v):
    raise NotImplementedError("write your pallas kernel here")


# speedup vs baseline: 1.4766x; 1.4766x over previous
import functools

import jax
import jax.numpy as jnp
from jax.experimental import pallas as pl
from jax.experimental.pallas import tpu as pltpu

_HI = jax.lax.Precision.HIGHEST


def _leaky(v):
    return jnp.maximum(v, 0.01 * v)


def _fold_bn(bias, g, b, m, v, eps=1e-5):
    s = g / jnp.sqrt(v + eps)
    t = (bias - m) * s + b
    return jnp.stack([s.astype(jnp.float32), t.astype(jnp.float32)], axis=0)


def _pick_tile(total, cands):
    for t in cands:
        if total % t == 0 and total // t >= 2:
            return t
    for t in cands:
        if total % t == 0:
            return t
    return total


def _nn_score_body(xall_ref, xrow_ref, o_ref, *, tn):
    xa = xall_ref[0]
    xr = xrow_ref[0]
    g = jax.lax.dot_general(xr, xa, (((0,), (0,)), ((), ())),
                            precision=_HI,
                            preferred_element_type=jnp.float32)
    col_sq = jnp.sum(xa * xa, axis=0, keepdims=True)
    score = col_sq - 2.0 * g
    row0 = pl.program_id(1) * tn
    rids = jax.lax.broadcasted_iota(jnp.int32, score.shape, 0) + row0
    cids = jax.lax.broadcasted_iota(jnp.int32, score.shape, 1)
    o_ref[0] = jnp.where(rids == cids, 1e30, score)


def _nn_indices(x, k):
    B, Fin, N = x.shape
    tn = _pick_tile(N, (1024, 512, 256, 128, 64, 32, 16, 8))
    scores = pl.pallas_call(
        functools.partial(_nn_score_body, tn=tn),
        out_shape=jax.ShapeDtypeStruct((B, N, N), jnp.float32),
        grid=(B, N // tn),
        in_specs=[
            pl.BlockSpec((1, Fin, N), lambda b, i: (b, 0, 0)),
            pl.BlockSpec((1, Fin, tn), lambda b, i: (b, 0, i)),
        ],
        out_specs=pl.BlockSpec((1, tn, N), lambda b, i: (b, i, 0)),
        compiler_params=pltpu.CompilerParams(
            dimension_semantics=("parallel", "parallel")),
    )(x, x)
    _, idx = jax.lax.top_k(-scores, k)
    return idx


def _head_body(xs_ref, w1_ref, st1_ref, w2_ref, st2_ref, o_ref):
    h = jnp.dot(xs_ref[...], w1_ref[...], precision=_HI,
                preferred_element_type=jnp.float32)
    h = _leaky(h * st1_ref[0:1, :] + st1_ref[1:2, :])
    y = jnp.dot(h, w2_ref[...], precision=_HI,
                preferred_element_type=jnp.float32)
    o_ref[...] = _leaky(y * st2_ref[0:1, :] + st2_ref[1:2, :])


def _edge_body(cen_ref, nbr_ref, ys_ref,
               wn_ref, wc_ref,
               wa1_ref, sta1_ref, wa2_ref, sta2_ref,
               w2_ref, st2_ref, stu_ref, o_ref, *, t, k, fin):
    f32 = jnp.float32
    cen = cen_ref[...]
    nbr = nbr_ref[0].reshape(t * k, cen.shape[1])

    zn = jnp.dot(nbr, wn_ref[...], preferred_element_type=f32)
    zc = jnp.dot(cen, wc_ref[...], preferred_element_type=f32)
    z = _leaky(zn.reshape(k, t, 96) + zc.reshape(1, t, 96))
    hk = z[..., :fin]
    w16 = z[..., fin:fin + 16] * z[..., fin + 16:fin + 32]

    w = w16.reshape(t * k, 16)
    w = jnp.dot(w, wa1_ref[...], preferred_element_type=f32)
    w = _leaky(w + sta1_ref[1:2, :])
    w = jnp.dot(w, wa2_ref[...], preferred_element_type=f32)
    w = _leaky(w + sta2_ref[1:2, :])
    w = w.reshape(k, t, fin)

    e = jnp.exp(w)
    r = 1.0 / jnp.sum(e, axis=0, keepdims=True)
    inte = hk * e * r

    y = jnp.dot(inte[0], w2_ref[0], preferred_element_type=f32)
    for j in range(1, k):
        y = y + jnp.dot(inte[j], w2_ref[j], preferred_element_type=f32)
    y = _leaky(y + st2_ref[1:2, :])
    y = _leaky(y * stu_ref[0:1, :] + stu_ref[1:2, :])

    yt = y.T
    head = jnp.broadcast_to(ys_ref[0], (fin, t))
    o_ref[0] = jnp.concatenate([head, yt], axis=0)


def kernel(x, pc,
           fea_w, fea_b, fea_bn_g, fea_bn_b, fea_bn_m, fea_bn_v,
           xyz_w, xyz_b, xyz_bn_g, xyz_bn_b, xyz_bn_m, xyz_bn_v,
           a1_w, a1_b, a1_bn_g, a1_bn_b, a1_bn_m, a1_bn_v,
           a2_w, a2_b, a2_bn_g, a2_bn_b, a2_bn_m, a2_bn_v,
           hk_w, hk_b, hk_bn_g, hk_bn_b, hk_bn_m, hk_bn_v,
           c2_w, c2_b, c2_bn_g, c2_bn_b, c2_bn_m, c2_bn_v,
           uc_bn_g, uc_bn_b, uc_bn_m, uc_bn_v,
           fc1_w, fc1_b, fb1_bn_g, fb1_bn_b, fb1_bn_m, fb1_bn_v,
           fc2_w, fc2_b, fb2_bn_g, fb2_bn_b, fb2_bn_m, fb2_bn_v):
    f32 = jnp.float32
    bf16 = jnp.bfloat16
    B, Fin, N = x.shape
    k = c2_w.shape[3]
    Fout = c2_w.shape[0]
    R = B * N
    C = Fin + 4

    idx = _nn_indices(x, k)

    x_cl = jnp.transpose(x, (0, 2, 1))
    pc_cl = jnp.transpose(pc, (0, 2, 1))
    xpc = jnp.concatenate(
        [x_cl, pc_cl, jnp.ones((B, N, 1), f32)], axis=2).astype(bf16)
    idx_t = jnp.transpose(idx, (0, 2, 1))
    nb = jax.vmap(lambda a, i: a[i])(xpc, idx_t.reshape(B, k * N))
    nbr4 = nb.reshape(B, k, N, C)
    cen2 = xpc.reshape(R, C)

    wfea = jnp.transpose(fea_w[:, :, 0, 0]).astype(f32)
    wxyz = jnp.transpose(xyz_w[:, :, 0, 0]).astype(f32)
    whk = jnp.transpose(hk_w[:, :, 0, 0]).astype(f32)
    wa1 = jnp.transpose(a1_w[:, :, 0, 0]).astype(f32)
    wa2 = jnp.transpose(a2_w[:, :, 0, 0]).astype(f32)
    w2m = jnp.transpose(c2_w[:, :, 0, :], (2, 1, 0))

    st_hk = _fold_bn(hk_b, hk_bn_g, hk_bn_b, hk_bn_m, hk_bn_v)
    st_fea = _fold_bn(fea_b, fea_bn_g, fea_bn_b, fea_bn_m, fea_bn_v)
    st_xyz = _fold_bn(xyz_b, xyz_bn_g, xyz_bn_b, xyz_bn_m, xyz_bn_v)
    stz = jnp.concatenate([st_hk, st_fea, st_xyz], axis=1)
    sta1 = _fold_bn(a1_b, a1_bn_g, a1_bn_b, a1_bn_m, a1_bn_v)
    sta2 = _fold_bn(a2_b, a2_bn_g, a2_bn_b, a2_bn_m, a2_bn_v)
    st2 = _fold_bn(c2_b, c2_bn_g, c2_bn_b, c2_bn_m, c2_bn_v)
    stu = _fold_bn(jnp.zeros((Fout,), f32), uc_bn_g, uc_bn_b, uc_bn_m, uc_bn_v)

    wn = jnp.zeros((C, 96), f32)
    wn = wn.at[:Fin, :Fin].set(whk[Fin:])
    wn = wn.at[:Fin, Fin:Fin + 16].set(wfea[Fin:])
    wn = wn.at[Fin:Fin + 3, Fin + 16:].set(wxyz[3:])
    wc = jnp.zeros((C, 96), f32)
    wc = wc.at[:Fin, :Fin].set(whk[:Fin])
    wc = wc.at[:Fin, Fin:Fin + 16].set(wfea[:Fin])
    wc = wc.at[Fin:Fin + 3, Fin + 16:].set(wxyz[:3])
    wcm = (wc - wn) * stz[0:1, :]
    wn = wn * stz[0:1, :]
    wn = wn.at[Fin + 3, :].set(stz[1])

    wa1 = wa1 * sta1[0:1, :]
    wa2 = wa2 * sta2[0:1, :]
    w2m = w2m * st2[0:1, :].reshape(1, 1, Fout)

    xs = jnp.max(x, axis=2)
    wf1 = jnp.transpose(fc1_w).astype(f32)
    stf1 = _fold_bn(fc1_b, fb1_bn_g, fb1_bn_b, fb1_bn_m, fb1_bn_v)
    wf2 = jnp.transpose(fc2_w).astype(f32)
    stf2 = _fold_bn(fc2_b, fb2_bn_g, fb2_bn_b, fb2_bn_m, fb2_bn_v)
    ys = pl.pallas_call(
        _head_body,
        out_shape=jax.ShapeDtypeStruct((B, Fout), f32),
        grid=(1,),
        in_specs=[
            pl.BlockSpec((B, Fin), lambda i: (0, 0)),
            pl.BlockSpec((Fin, Fin), lambda i: (0, 0)),
            pl.BlockSpec((2, Fin), lambda i: (0, 0)),
            pl.BlockSpec((Fin, Fout), lambda i: (0, 0)),
            pl.BlockSpec((2, Fout), lambda i: (0, 0)),
        ],
        out_specs=pl.BlockSpec((B, Fout), lambda i: (0, 0)),
        compiler_params=pltpu.CompilerParams(
            dimension_semantics=("arbitrary",)),
    )(xs, wf1, stf1, wf2, stf2)
    ys3 = ys.reshape(B, Fout, 1)

    T = _pick_tile(N, (256, 128, 64, 32, 16, 8))
    nper = N // T
    const2 = lambda r: (0, 0)
    out = pl.pallas_call(
        functools.partial(_edge_body, t=T, k=k, fin=Fin),
        out_shape=jax.ShapeDtypeStruct((B, 2 * Fout, N), f32),
        grid=(R // T,),
        in_specs=[
            pl.BlockSpec((T, C), lambda r: (r, 0)),
            pl.BlockSpec((1, k, T, C), lambda r, m=nper: (r // m, 0, r % m, 0)),
            pl.BlockSpec((1, Fout, 1), lambda r, m=nper: (r // m, 0, 0)),
            pl.BlockSpec((C, 96), const2),
            pl.BlockSpec((C, 96), const2),
            pl.BlockSpec((16, 64), const2),
            pl.BlockSpec((2, 64), const2),
            pl.BlockSpec((64, Fin), const2),
            pl.BlockSpec((2, Fin), const2),
            pl.BlockSpec((k, Fin, Fout), lambda r: (0, 0, 0)),
            pl.BlockSpec((2, Fout), const2),
            pl.BlockSpec((2, Fout), const2),
        ],
        out_specs=pl.BlockSpec((1, 2 * Fout, T),
                               lambda r, m=nper: (r // m, 0, r % m)),
        compiler_params=pltpu.CompilerParams(
            dimension_semantics=("parallel",)),
    )(cen2, nbr4, ys3,
      wn.astype(bf16), wcm.astype(bf16),
      wa1, sta1, wa2, sta2,
      w2m, st2, stu)
    return out


# speedup vs baseline: 13.7230x; 9.2935x over previous
import functools

import jax
import jax.numpy as jnp
from jax.experimental import pallas as pl
from jax.experimental.pallas import tpu as pltpu

_HI = jax.lax.Precision.HIGHEST


def _leaky(v):
    return jnp.maximum(v, 0.01 * v)


def _fold_bn(bias, g, b, m, v, eps=1e-5):
    s = g / jnp.sqrt(v + eps)
    t = (bias - m) * s + b
    return jnp.stack([s.astype(jnp.float32), t.astype(jnp.float32)], axis=0)


def _pick_tile(total, cands):
    for t in cands:
        if total % t == 0 and total // t >= 2:
            return t
    for t in cands:
        if total % t == 0:
            return t
    return total


def _nn_topk_body(xall_ref, xrow_ref, o_ref, *, tn, k):
    xa = xall_ref[0]
    xr = xrow_ref[0]
    g = jax.lax.dot_general(xr, xa, (((0,), (0,)), ((), ())),
                            precision=_HI,
                            preferred_element_type=jnp.float32)
    col_sq = jnp.sum(xa * xa, axis=0, keepdims=True)
    score = col_sq - 2.0 * g
    row0 = pl.program_id(1) * tn
    rids = jax.lax.broadcasted_iota(jnp.int32, score.shape, 0) + row0
    cids = jax.lax.broadcasted_iota(jnp.int32, score.shape, 1)
    score = jnp.where(rids == cids, 1e30, score)

    outs = []
    for _ in range(k):
        m = jnp.min(score, axis=1, keepdims=True)
        am = jnp.min(jnp.where(score == m, cids, jnp.int32(1 << 30)),
                     axis=1, keepdims=True)
        outs.append(am)
        score = jnp.where(cids == am, 1e30, score)
    o_ref[0] = jnp.concatenate(outs, axis=1)


def _nn_indices(x, k):
    B, Fin, N = x.shape
    tn = _pick_tile(N, (256, 128, 64, 32, 16, 8))
    return pl.pallas_call(
        functools.partial(_nn_topk_body, tn=tn, k=k),
        out_shape=jax.ShapeDtypeStruct((B, N, k), jnp.int32),
        grid=(B, N // tn),
        in_specs=[
            pl.BlockSpec((1, Fin, N), lambda b, i: (b, 0, 0)),
            pl.BlockSpec((1, Fin, tn), lambda b, i: (b, 0, i)),
        ],
        out_specs=pl.BlockSpec((1, tn, k), lambda b, i: (b, i, 0)),
        compiler_params=pltpu.CompilerParams(
            dimension_semantics=("parallel", "parallel")),
    )(x, x)


def _head_body(xs_ref, w1_ref, st1_ref, w2_ref, st2_ref, o_ref):
    h = jnp.dot(xs_ref[...], w1_ref[...], precision=_HI,
                preferred_element_type=jnp.float32)
    h = _leaky(h * st1_ref[0:1, :] + st1_ref[1:2, :])
    y = jnp.dot(h, w2_ref[...], precision=_HI,
                preferred_element_type=jnp.float32)
    o_ref[...] = _leaky(y * st2_ref[0:1, :] + st2_ref[1:2, :])


def _edge_body(cen_ref, xsrc_ref, idx_ref, ys_ref,
               wn_ref, wc_ref,
               wa1_ref, sta1_ref, wa2_ref, sta2_ref,
               w2_ref, st2_ref, stu_ref, o_ref, gbuf, *, t, k, fin):
    f32 = jnp.float32
    cen = cen_ref[...]

    def _gather8(i, carry):
        t0 = i * 8
        for u in range(8):
            for j in range(k):
                s = idx_ref[t0 + u, j]
                gbuf[j * t + t0 + u, :] = xsrc_ref[s, 0, :]
        return carry
    jax.lax.fori_loop(0, t // 8, _gather8, 0)
    nbr = gbuf[...]

    zn = jnp.dot(nbr, wn_ref[...], preferred_element_type=f32)
    zc = jnp.dot(cen, wc_ref[...], preferred_element_type=f32)
    z = _leaky(zn.reshape(k, t, 96) + zc.reshape(1, t, 96))
    hk = z[..., :fin]
    w16 = z[..., fin:fin + 16] * z[..., fin + 16:fin + 32]

    w = w16.reshape(t * k, 16)
    w = jnp.dot(w, wa1_ref[...], preferred_element_type=f32)
    w = _leaky(w + sta1_ref[1:2, :])
    w = jnp.dot(w, wa2_ref[...], preferred_element_type=f32)
    w = _leaky(w + sta2_ref[1:2, :])
    w = w.reshape(k, t, fin)

    e = jnp.exp(w)
    r = 1.0 / jnp.sum(e, axis=0, keepdims=True)
    inte = hk * e * r

    y = jnp.dot(inte[0], w2_ref[0], preferred_element_type=f32)
    for j in range(1, k):
        y = y + jnp.dot(inte[j], w2_ref[j], preferred_element_type=f32)
    y = _leaky(y + st2_ref[1:2, :])
    y = _leaky(y * stu_ref[0:1, :] + stu_ref[1:2, :])

    yt = y.T
    head = jnp.broadcast_to(ys_ref[0], (fin, t))
    o_ref[0] = jnp.concatenate([head, yt], axis=0)


def kernel(x, pc,
           fea_w, fea_b, fea_bn_g, fea_bn_b, fea_bn_m, fea_bn_v,
           xyz_w, xyz_b, xyz_bn_g, xyz_bn_b, xyz_bn_m, xyz_bn_v,
           a1_w, a1_b, a1_bn_g, a1_bn_b, a1_bn_m, a1_bn_v,
           a2_w, a2_b, a2_bn_g, a2_bn_b, a2_bn_m, a2_bn_v,
           hk_w, hk_b, hk_bn_g, hk_bn_b, hk_bn_m, hk_bn_v,
           c2_w, c2_b, c2_bn_g, c2_bn_b, c2_bn_m, c2_bn_v,
           uc_bn_g, uc_bn_b, uc_bn_m, uc_bn_v,
           fc1_w, fc1_b, fb1_bn_g, fb1_bn_b, fb1_bn_m, fb1_bn_v,
           fc2_w, fc2_b, fb2_bn_g, fb2_bn_b, fb2_bn_m, fb2_bn_v):
    f32 = jnp.float32
    bf16 = jnp.bfloat16
    B, Fin, N = x.shape
    k = c2_w.shape[3]
    Fout = c2_w.shape[0]
    R = B * N
    C = Fin + 4

    idx = _nn_indices(x, k)

    x_cl = jnp.transpose(x, (0, 2, 1))
    pc_cl = jnp.transpose(pc, (0, 2, 1))
    xpc = jnp.concatenate(
        [x_cl, pc_cl, jnp.ones((B, N, 1), f32)], axis=2)
    cen2 = xpc.reshape(R, C)
    xpc3 = xpc.reshape(R, 1, C)
    idx2 = idx.reshape(R, k)

    wfea = jnp.transpose(fea_w[:, :, 0, 0]).astype(f32)
    wxyz = jnp.transpose(xyz_w[:, :, 0, 0]).astype(f32)
    whk = jnp.transpose(hk_w[:, :, 0, 0]).astype(f32)
    wa1 = jnp.transpose(a1_w[:, :, 0, 0]).astype(f32)
    wa2 = jnp.transpose(a2_w[:, :, 0, 0]).astype(f32)
    w2m = jnp.transpose(c2_w[:, :, 0, :], (2, 1, 0))

    st_hk = _fold_bn(hk_b, hk_bn_g, hk_bn_b, hk_bn_m, hk_bn_v)
    st_fea = _fold_bn(fea_b, fea_bn_g, fea_bn_b, fea_bn_m, fea_bn_v)
    st_xyz = _fold_bn(xyz_b, xyz_bn_g, xyz_bn_b, xyz_bn_m, xyz_bn_v)
    stz = jnp.concatenate([st_hk, st_fea, st_xyz], axis=1)
    sta1 = _fold_bn(a1_b, a1_bn_g, a1_bn_b, a1_bn_m, a1_bn_v)
    sta2 = _fold_bn(a2_b, a2_bn_g, a2_bn_b, a2_bn_m, a2_bn_v)
    st2 = _fold_bn(c2_b, c2_bn_g, c2_bn_b, c2_bn_m, c2_bn_v)
    stu = _fold_bn(jnp.zeros((Fout,), f32), uc_bn_g, uc_bn_b, uc_bn_m, uc_bn_v)

    wn = jnp.zeros((C, 96), f32)
    wn = wn.at[:Fin, :Fin].set(whk[Fin:])
    wn = wn.at[:Fin, Fin:Fin + 16].set(wfea[Fin:])
    wn = wn.at[Fin:Fin + 3, Fin + 16:].set(wxyz[3:])
    wc = jnp.zeros((C, 96), f32)
    wc = wc.at[:Fin, :Fin].set(whk[:Fin])
    wc = wc.at[:Fin, Fin:Fin + 16].set(wfea[:Fin])
    wc = wc.at[Fin:Fin + 3, Fin + 16:].set(wxyz[:3])
    wcm = (wc - wn) * stz[0:1, :]
    wn = wn * stz[0:1, :]
    wn = wn.at[Fin + 3, :].set(stz[1])

    wa1 = wa1 * sta1[0:1, :]
    wa2 = wa2 * sta2[0:1, :]
    w2m = w2m * st2[0:1, :].reshape(1, 1, Fout)

    xs = jnp.max(x, axis=2)
    wf1 = jnp.transpose(fc1_w).astype(f32)
    stf1 = _fold_bn(fc1_b, fb1_bn_g, fb1_bn_b, fb1_bn_m, fb1_bn_v)
    wf2 = jnp.transpose(fc2_w).astype(f32)
    stf2 = _fold_bn(fc2_b, fb2_bn_g, fb2_bn_b, fb2_bn_m, fb2_bn_v)
    ys = pl.pallas_call(
        _head_body,
        out_shape=jax.ShapeDtypeStruct((B, Fout), f32),
        grid=(1,),
        in_specs=[
            pl.BlockSpec((B, Fin), lambda i: (0, 0)),
            pl.BlockSpec((Fin, Fin), lambda i: (0, 0)),
            pl.BlockSpec((2, Fin), lambda i: (0, 0)),
            pl.BlockSpec((Fin, Fout), lambda i: (0, 0)),
            pl.BlockSpec((2, Fout), lambda i: (0, 0)),
        ],
        out_specs=pl.BlockSpec((B, Fout), lambda i: (0, 0)),
        compiler_params=pltpu.CompilerParams(
            dimension_semantics=("arbitrary",)),
    )(xs, wf1, stf1, wf2, stf2)
    ys3 = ys.reshape(B, Fout, 1)

    T = _pick_tile(N, (256, 128, 64, 32, 16, 8))
    nper = N // T
    const2 = lambda r: (0, 0)
    out = pl.pallas_call(
        functools.partial(_edge_body, t=T, k=k, fin=Fin),
        out_shape=jax.ShapeDtypeStruct((B, 2 * Fout, N), f32),
        grid=(R // T,),
        in_specs=[
            pl.BlockSpec((T, C), lambda r: (r, 0)),
            pl.BlockSpec((N, 1, C), lambda r, m=nper: (r // m, 0, 0)),
            pl.BlockSpec((T, k), lambda r: (r, 0), memory_space=pltpu.SMEM),
            pl.BlockSpec((1, Fout, 1), lambda r, m=nper: (r // m, 0, 0)),
            pl.BlockSpec((C, 96), const2),
            pl.BlockSpec((C, 96), const2),
            pl.BlockSpec((16, 64), const2),
            pl.BlockSpec((2, 64), const2),
            pl.BlockSpec((64, Fin), const2),
            pl.BlockSpec((2, Fin), const2),
            pl.BlockSpec((k, Fin, Fout), lambda r: (0, 0, 0)),
            pl.BlockSpec((2, Fout), const2),
            pl.BlockSpec((2, Fout), const2),
        ],
        out_specs=pl.BlockSpec((1, 2 * Fout, T),
                               lambda r, m=nper: (r // m, 0, r % m)),
        scratch_shapes=[pltpu.VMEM((k * T, C), f32)],
        compiler_params=pltpu.CompilerParams(
            dimension_semantics=("parallel",)),
    )(cen2, xpc3, idx2, ys3,
      wn, wcm,
      wa1, sta1, wa2, sta2,
      w2m, st2, stu)
    return out


# speedup vs baseline: 14.8091x; 1.0791x over previous
import functools

import jax
import jax.numpy as jnp
from jax.experimental import pallas as pl
from jax.experimental.pallas import tpu as pltpu

_HI = jax.lax.Precision.HIGHEST


def _leaky(v):
    return jnp.maximum(v, 0.01 * v)


def _fold_bn(bias, g, b, m, v, eps=1e-5):
    s = g / jnp.sqrt(v + eps)
    t = (bias - m) * s + b
    return jnp.stack([s.astype(jnp.float32), t.astype(jnp.float32)], axis=0)


def _pick_tile(total, cands):
    for t in cands:
        if total % t == 0 and total // t >= 2:
            return t
    for t in cands:
        if total % t == 0:
            return t
    return total


def _nn_topk_body(xall_ref, xrow_ref, o_ref, *, tn, k):
    xa = xall_ref[0]
    xr = xrow_ref[0]
    g = jax.lax.dot_general(xr, xa, (((0,), (0,)), ((), ())),
                            precision=_HI,
                            preferred_element_type=jnp.float32)
    col_sq = jnp.sum(xa * xa, axis=0, keepdims=True)
    score = col_sq - 2.0 * g
    row0 = pl.program_id(1) * tn
    rids = jax.lax.broadcasted_iota(jnp.int32, score.shape, 0) + row0
    cids = jax.lax.broadcasted_iota(jnp.int32, score.shape, 1)
    score = jnp.where(rids == cids, 1e30, score)

    outs = []
    for _ in range(k):
        m = jnp.min(score, axis=1, keepdims=True)
        am = jnp.min(jnp.where(score == m, cids, jnp.int32(1 << 30)),
                     axis=1, keepdims=True)
        outs.append(am)
        score = jnp.where(cids == am, 1e30, score)
    o_ref[0] = jnp.concatenate(outs, axis=1)


def _nn_indices(x, k):
    B, Fin, N = x.shape
    tn = _pick_tile(N, (512, 256, 128, 64, 32, 16, 8))
    return pl.pallas_call(
        functools.partial(_nn_topk_body, tn=tn, k=k),
        out_shape=jax.ShapeDtypeStruct((B, N, k), jnp.int32),
        grid=(B, N // tn),
        in_specs=[
            pl.BlockSpec((1, Fin, N), lambda b, i: (b, 0, 0)),
            pl.BlockSpec((1, Fin, tn), lambda b, i: (b, 0, i)),
        ],
        out_specs=pl.BlockSpec((1, tn, k), lambda b, i: (b, i, 0)),
        compiler_params=pltpu.CompilerParams(
            dimension_semantics=("parallel", "parallel")),
    )(x, x)


def _head_body(xs_ref, w1_ref, st1_ref, w2_ref, st2_ref, o_ref):
    h = jnp.dot(xs_ref[...], w1_ref[...], precision=_HI,
                preferred_element_type=jnp.float32)
    h = _leaky(h * st1_ref[0:1, :] + st1_ref[1:2, :])
    y = jnp.dot(h, w2_ref[...], precision=_HI,
                preferred_element_type=jnp.float32)
    o_ref[...] = _leaky(y * st2_ref[0:1, :] + st2_ref[1:2, :])


def _edge_body(cen_ref, xsrc_ref, idx_ref, ys_ref,
               wn_ref, wc_ref,
               wa1_ref, sta1_ref, wa2_ref, sta2_ref,
               w2_ref, st2_ref, stu_ref, o_ref, gbuf, *, t, k, fin):
    f32 = jnp.float32
    cen = cen_ref[...]

    def _gather8(i, carry):
        t0 = i * 8
        for u in range(8):
            for j in range(k):
                s = idx_ref[t0 + u, j]
                gbuf[j * t + t0 + u, :] = xsrc_ref[s, 0, :]
        return carry
    jax.lax.fori_loop(0, t // 8, _gather8, 0)
    nbr = gbuf[...]

    zn = jnp.dot(nbr, wn_ref[...], preferred_element_type=f32)
    zc = jnp.dot(cen, wc_ref[...], preferred_element_type=f32)
    z = _leaky(zn.reshape(k, t, 96) + zc.reshape(1, t, 96))
    hk = z[..., :fin]
    w16 = z[..., fin:fin + 16] * z[..., fin + 16:fin + 32]

    w = w16.reshape(t * k, 16)
    w = jnp.dot(w, wa1_ref[...], preferred_element_type=f32)
    w = _leaky(w + sta1_ref[1:2, :])
    w = jnp.dot(w, wa2_ref[...], preferred_element_type=f32)
    w = _leaky(w + sta2_ref[1:2, :])
    w = w.reshape(k, t, fin)

    e = jnp.exp(w)
    r = 1.0 / jnp.sum(e, axis=0, keepdims=True)
    inte = hk * e * r

    y = jnp.dot(inte[0], w2_ref[0], preferred_element_type=f32)
    for j in range(1, k):
        y = y + jnp.dot(inte[j], w2_ref[j], preferred_element_type=f32)
    y = _leaky(y + st2_ref[1:2, :])
    y = _leaky(y * stu_ref[0:1, :] + stu_ref[1:2, :])

    yt = y.T
    head = jnp.broadcast_to(ys_ref[0], (fin, t))
    o_ref[0] = jnp.concatenate([head, yt], axis=0)


def kernel(x, pc,
           fea_w, fea_b, fea_bn_g, fea_bn_b, fea_bn_m, fea_bn_v,
           xyz_w, xyz_b, xyz_bn_g, xyz_bn_b, xyz_bn_m, xyz_bn_v,
           a1_w, a1_b, a1_bn_g, a1_bn_b, a1_bn_m, a1_bn_v,
           a2_w, a2_b, a2_bn_g, a2_bn_b, a2_bn_m, a2_bn_v,
           hk_w, hk_b, hk_bn_g, hk_bn_b, hk_bn_m, hk_bn_v,
           c2_w, c2_b, c2_bn_g, c2_bn_b, c2_bn_m, c2_bn_v,
           uc_bn_g, uc_bn_b, uc_bn_m, uc_bn_v,
           fc1_w, fc1_b, fb1_bn_g, fb1_bn_b, fb1_bn_m, fb1_bn_v,
           fc2_w, fc2_b, fb2_bn_g, fb2_bn_b, fb2_bn_m, fb2_bn_v):
    f32 = jnp.float32
    bf16 = jnp.bfloat16
    B, Fin, N = x.shape
    k = c2_w.shape[3]
    Fout = c2_w.shape[0]
    R = B * N
    C = Fin + 4

    idx = _nn_indices(x, k)

    x_cl = jnp.transpose(x, (0, 2, 1))
    pc_cl = jnp.transpose(pc, (0, 2, 1))
    xpc = jnp.concatenate(
        [x_cl, pc_cl, jnp.ones((B, N, 1), f32)], axis=2)
    cen2 = xpc.reshape(R, C)
    xpc3 = xpc.reshape(R, 1, C)
    idx2 = idx.reshape(R, k)

    wfea = jnp.transpose(fea_w[:, :, 0, 0]).astype(f32)
    wxyz = jnp.transpose(xyz_w[:, :, 0, 0]).astype(f32)
    whk = jnp.transpose(hk_w[:, :, 0, 0]).astype(f32)
    wa1 = jnp.transpose(a1_w[:, :, 0, 0]).astype(f32)
    wa2 = jnp.transpose(a2_w[:, :, 0, 0]).astype(f32)
    w2m = jnp.transpose(c2_w[:, :, 0, :], (2, 1, 0))

    st_hk = _fold_bn(hk_b, hk_bn_g, hk_bn_b, hk_bn_m, hk_bn_v)
    st_fea = _fold_bn(fea_b, fea_bn_g, fea_bn_b, fea_bn_m, fea_bn_v)
    st_xyz = _fold_bn(xyz_b, xyz_bn_g, xyz_bn_b, xyz_bn_m, xyz_bn_v)
    stz = jnp.concatenate([st_hk, st_fea, st_xyz], axis=1)
    sta1 = _fold_bn(a1_b, a1_bn_g, a1_bn_b, a1_bn_m, a1_bn_v)
    sta2 = _fold_bn(a2_b, a2_bn_g, a2_bn_b, a2_bn_m, a2_bn_v)
    st2 = _fold_bn(c2_b, c2_bn_g, c2_bn_b, c2_bn_m, c2_bn_v)
    stu = _fold_bn(jnp.zeros((Fout,), f32), uc_bn_g, uc_bn_b, uc_bn_m, uc_bn_v)

    wn = jnp.zeros((C, 96), f32)
    wn = wn.at[:Fin, :Fin].set(whk[Fin:])
    wn = wn.at[:Fin, Fin:Fin + 16].set(wfea[Fin:])
    wn = wn.at[Fin:Fin + 3, Fin + 16:].set(wxyz[3:])
    wc = jnp.zeros((C, 96), f32)
    wc = wc.at[:Fin, :Fin].set(whk[:Fin])
    wc = wc.at[:Fin, Fin:Fin + 16].set(wfea[:Fin])
    wc = wc.at[Fin:Fin + 3, Fin + 16:].set(wxyz[:3])
    wcm = (wc - wn) * stz[0:1, :]
    wn = wn * stz[0:1, :]
    wn = wn.at[Fin + 3, :].set(stz[1])

    wa1 = wa1 * sta1[0:1, :]
    wa2 = wa2 * sta2[0:1, :]
    w2m = w2m * st2[0:1, :].reshape(1, 1, Fout)

    xs = jnp.max(x, axis=2)
    wf1 = jnp.transpose(fc1_w).astype(f32)
    stf1 = _fold_bn(fc1_b, fb1_bn_g, fb1_bn_b, fb1_bn_m, fb1_bn_v)
    wf2 = jnp.transpose(fc2_w).astype(f32)
    stf2 = _fold_bn(fc2_b, fb2_bn_g, fb2_bn_b, fb2_bn_m, fb2_bn_v)
    ys = pl.pallas_call(
        _head_body,
        out_shape=jax.ShapeDtypeStruct((B, Fout), f32),
        grid=(1,),
        in_specs=[
            pl.BlockSpec((B, Fin), lambda i: (0, 0)),
            pl.BlockSpec((Fin, Fin), lambda i: (0, 0)),
            pl.BlockSpec((2, Fin), lambda i: (0, 0)),
            pl.BlockSpec((Fin, Fout), lambda i: (0, 0)),
            pl.BlockSpec((2, Fout), lambda i: (0, 0)),
        ],
        out_specs=pl.BlockSpec((B, Fout), lambda i: (0, 0)),
        compiler_params=pltpu.CompilerParams(
            dimension_semantics=("arbitrary",)),
    )(xs, wf1, stf1, wf2, stf2)
    ys3 = ys.reshape(B, Fout, 1)

    T = _pick_tile(N, (512, 256, 128, 64, 32, 16, 8))
    nper = N // T
    const2 = lambda r: (0, 0)
    out = pl.pallas_call(
        functools.partial(_edge_body, t=T, k=k, fin=Fin),
        out_shape=jax.ShapeDtypeStruct((B, 2 * Fout, N), f32),
        grid=(R // T,),
        in_specs=[
            pl.BlockSpec((T, C), lambda r: (r, 0)),
            pl.BlockSpec((N, 1, C), lambda r, m=nper: (r // m, 0, 0)),
            pl.BlockSpec((T, k), lambda r: (r, 0), memory_space=pltpu.SMEM),
            pl.BlockSpec((1, Fout, 1), lambda r, m=nper: (r // m, 0, 0)),
            pl.BlockSpec((C, 96), const2),
            pl.BlockSpec((C, 96), const2),
            pl.BlockSpec((16, 64), const2),
            pl.BlockSpec((2, 64), const2),
            pl.BlockSpec((64, Fin), const2),
            pl.BlockSpec((2, Fin), const2),
            pl.BlockSpec((k, Fin, Fout), lambda r: (0, 0, 0)),
            pl.BlockSpec((2, Fout), const2),
            pl.BlockSpec((2, Fout), const2),
        ],
        out_specs=pl.BlockSpec((1, 2 * Fout, T),
                               lambda r, m=nper: (r // m, 0, r % m)),
        scratch_shapes=[pltpu.VMEM((k * T, C), f32)],
        compiler_params=pltpu.CompilerParams(
            dimension_semantics=("parallel",)),
    )(cen2, xpc3, idx2, ys3,
      wn, wcm,
      wa1, sta1, wa2, sta2,
      w2m, st2, stu)
    return out


# speedup vs baseline: 16.2342x; 1.0962x over previous
import functools

import jax
import jax.numpy as jnp
from jax.experimental import pallas as pl
from jax.experimental.pallas import tpu as pltpu

_HI = jax.lax.Precision.HIGHEST


def _leaky(v):
    return jnp.maximum(v, 0.01 * v)


def _fold_bn(bias, g, b, m, v, eps=1e-5):
    s = g / jnp.sqrt(v + eps)
    t = (bias - m) * s + b
    return jnp.stack([s.astype(jnp.float32), t.astype(jnp.float32)], axis=0)


def _pick_tile(total, cands):
    for t in cands:
        if total % t == 0 and total // t >= 2:
            return t
    for t in cands:
        if total % t == 0:
            return t
    return total


def _nn_topk_body(xall_ref, xrow_ref, o_ref, *, tn, k):
    xa = xall_ref[0]
    xr = xrow_ref[0]
    g = jax.lax.dot_general(xr, xa, (((0,), (0,)), ((), ())),
                            precision=_HI,
                            preferred_element_type=jnp.float32)
    col_sq = jnp.sum(xa * xa, axis=0, keepdims=True)
    score = col_sq - 2.0 * g
    row0 = pl.program_id(1) * tn
    rids = jax.lax.broadcasted_iota(jnp.int32, score.shape, 0) + row0
    cids = jax.lax.broadcasted_iota(jnp.int32, score.shape, 1)
    score = jnp.where(rids == cids, 1e30, score)

    outs = []
    for _ in range(k):
        am = jnp.argmin(score, axis=1).astype(jnp.int32)[:, None]
        outs.append(am)
        score = jnp.where(cids == am, 1e30, score)
    o_ref[0] = jnp.concatenate(outs, axis=1)


def _nn_indices(x, k):
    B, Fin, N = x.shape
    tn = _pick_tile(N, (512, 256, 128, 64, 32, 16, 8))
    return pl.pallas_call(
        functools.partial(_nn_topk_body, tn=tn, k=k),
        out_shape=jax.ShapeDtypeStruct((B, N, k), jnp.int32),
        grid=(B, N // tn),
        in_specs=[
            pl.BlockSpec((1, Fin, N), lambda b, i: (b, 0, 0)),
            pl.BlockSpec((1, Fin, tn), lambda b, i: (b, 0, i)),
        ],
        out_specs=pl.BlockSpec((1, tn, k), lambda b, i: (b, i, 0)),
        compiler_params=pltpu.CompilerParams(
            dimension_semantics=("parallel", "parallel")),
    )(x, x)


def _head_body(xs_ref, w1_ref, st1_ref, w2_ref, st2_ref, o_ref):
    h = jnp.dot(xs_ref[...], w1_ref[...], precision=_HI,
                preferred_element_type=jnp.float32)
    h = _leaky(h * st1_ref[0:1, :] + st1_ref[1:2, :])
    y = jnp.dot(h, w2_ref[...], precision=_HI,
                preferred_element_type=jnp.float32)
    o_ref[...] = _leaky(y * st2_ref[0:1, :] + st2_ref[1:2, :])


def _edge_body(cen_ref, xsrc_ref, idx_ref, ys_ref,
               wn_ref, wc_ref,
               wa1_ref, sta1_ref, wa2_ref, sta2_ref,
               w2_ref, st2_ref, stu_ref, o_ref, gbuf, *, t, k, fin):
    f32 = jnp.float32
    cen = cen_ref[...]

    def _gather8(i, carry):
        t0 = i * 8
        for u in range(8):
            for j in range(k):
                s = idx_ref[t0 + u, j]
                gbuf[j * t + t0 + u, :] = xsrc_ref[s, 0, :]
        return carry
    jax.lax.fori_loop(0, t // 8, _gather8, 0)
    nbr = gbuf[...]

    zn = jnp.dot(nbr, wn_ref[...], preferred_element_type=f32)
    zc = jnp.dot(cen, wc_ref[...], preferred_element_type=f32)
    z = _leaky(zn.reshape(k, t, 96) + zc.reshape(1, t, 96))
    hk = z[..., :fin]
    w16 = z[..., fin:fin + 16] * z[..., fin + 16:fin + 32]

    w = w16.reshape(t * k, 16)
    w = jnp.dot(w, wa1_ref[...], preferred_element_type=f32)
    w = _leaky(w + sta1_ref[1:2, :])
    w = jnp.dot(w, wa2_ref[...], preferred_element_type=f32)
    w = _leaky(w + sta2_ref[1:2, :])
    w = w.reshape(k, t, fin)

    e = jnp.exp(w)
    r = 1.0 / jnp.sum(e, axis=0, keepdims=True)
    inte = hk * e * r

    y = jnp.dot(inte[0], w2_ref[0], preferred_element_type=f32)
    for j in range(1, k):
        y = y + jnp.dot(inte[j], w2_ref[j], preferred_element_type=f32)
    y = _leaky(y + st2_ref[1:2, :])
    y = _leaky(y * stu_ref[0:1, :] + stu_ref[1:2, :])

    yt = y.T
    head = jnp.broadcast_to(ys_ref[0], (fin, t))
    o_ref[0] = jnp.concatenate([head, yt], axis=0)


def kernel(x, pc,
           fea_w, fea_b, fea_bn_g, fea_bn_b, fea_bn_m, fea_bn_v,
           xyz_w, xyz_b, xyz_bn_g, xyz_bn_b, xyz_bn_m, xyz_bn_v,
           a1_w, a1_b, a1_bn_g, a1_bn_b, a1_bn_m, a1_bn_v,
           a2_w, a2_b, a2_bn_g, a2_bn_b, a2_bn_m, a2_bn_v,
           hk_w, hk_b, hk_bn_g, hk_bn_b, hk_bn_m, hk_bn_v,
           c2_w, c2_b, c2_bn_g, c2_bn_b, c2_bn_m, c2_bn_v,
           uc_bn_g, uc_bn_b, uc_bn_m, uc_bn_v,
           fc1_w, fc1_b, fb1_bn_g, fb1_bn_b, fb1_bn_m, fb1_bn_v,
           fc2_w, fc2_b, fb2_bn_g, fb2_bn_b, fb2_bn_m, fb2_bn_v):
    f32 = jnp.float32
    bf16 = jnp.bfloat16
    B, Fin, N = x.shape
    k = c2_w.shape[3]
    Fout = c2_w.shape[0]
    R = B * N
    C = Fin + 4

    idx = _nn_indices(x, k)

    x_cl = jnp.transpose(x, (0, 2, 1))
    pc_cl = jnp.transpose(pc, (0, 2, 1))
    xpc = jnp.concatenate(
        [x_cl, pc_cl, jnp.ones((B, N, 1), f32)], axis=2)
    cen2 = xpc.reshape(R, C)
    xpc3 = xpc.reshape(R, 1, C)
    idx2 = idx.reshape(R, k)

    wfea = jnp.transpose(fea_w[:, :, 0, 0]).astype(f32)
    wxyz = jnp.transpose(xyz_w[:, :, 0, 0]).astype(f32)
    whk = jnp.transpose(hk_w[:, :, 0, 0]).astype(f32)
    wa1 = jnp.transpose(a1_w[:, :, 0, 0]).astype(f32)
    wa2 = jnp.transpose(a2_w[:, :, 0, 0]).astype(f32)
    w2m = jnp.transpose(c2_w[:, :, 0, :], (2, 1, 0))

    st_hk = _fold_bn(hk_b, hk_bn_g, hk_bn_b, hk_bn_m, hk_bn_v)
    st_fea = _fold_bn(fea_b, fea_bn_g, fea_bn_b, fea_bn_m, fea_bn_v)
    st_xyz = _fold_bn(xyz_b, xyz_bn_g, xyz_bn_b, xyz_bn_m, xyz_bn_v)
    stz = jnp.concatenate([st_hk, st_fea, st_xyz], axis=1)
    sta1 = _fold_bn(a1_b, a1_bn_g, a1_bn_b, a1_bn_m, a1_bn_v)
    sta2 = _fold_bn(a2_b, a2_bn_g, a2_bn_b, a2_bn_m, a2_bn_v)
    st2 = _fold_bn(c2_b, c2_bn_g, c2_bn_b, c2_bn_m, c2_bn_v)
    stu = _fold_bn(jnp.zeros((Fout,), f32), uc_bn_g, uc_bn_b, uc_bn_m, uc_bn_v)

    wn = jnp.zeros((C, 96), f32)
    wn = wn.at[:Fin, :Fin].set(whk[Fin:])
    wn = wn.at[:Fin, Fin:Fin + 16].set(wfea[Fin:])
    wn = wn.at[Fin:Fin + 3, Fin + 16:].set(wxyz[3:])
    wc = jnp.zeros((C, 96), f32)
    wc = wc.at[:Fin, :Fin].set(whk[:Fin])
    wc = wc.at[:Fin, Fin:Fin + 16].set(wfea[:Fin])
    wc = wc.at[Fin:Fin + 3, Fin + 16:].set(wxyz[:3])
    wcm = (wc - wn) * stz[0:1, :]
    wn = wn * stz[0:1, :]
    wn = wn.at[Fin + 3, :].set(stz[1])

    wa1 = wa1 * sta1[0:1, :]
    wa2 = wa2 * sta2[0:1, :]
    w2m = w2m * st2[0:1, :].reshape(1, 1, Fout)

    xs = jnp.max(x, axis=2)
    wf1 = jnp.transpose(fc1_w).astype(f32)
    stf1 = _fold_bn(fc1_b, fb1_bn_g, fb1_bn_b, fb1_bn_m, fb1_bn_v)
    wf2 = jnp.transpose(fc2_w).astype(f32)
    stf2 = _fold_bn(fc2_b, fb2_bn_g, fb2_bn_b, fb2_bn_m, fb2_bn_v)
    ys = pl.pallas_call(
        _head_body,
        out_shape=jax.ShapeDtypeStruct((B, Fout), f32),
        grid=(1,),
        in_specs=[
            pl.BlockSpec((B, Fin), lambda i: (0, 0)),
            pl.BlockSpec((Fin, Fin), lambda i: (0, 0)),
            pl.BlockSpec((2, Fin), lambda i: (0, 0)),
            pl.BlockSpec((Fin, Fout), lambda i: (0, 0)),
            pl.BlockSpec((2, Fout), lambda i: (0, 0)),
        ],
        out_specs=pl.BlockSpec((B, Fout), lambda i: (0, 0)),
        compiler_params=pltpu.CompilerParams(
            dimension_semantics=("arbitrary",)),
    )(xs, wf1, stf1, wf2, stf2)
    ys3 = ys.reshape(B, Fout, 1)

    T = _pick_tile(N, (512, 256, 128, 64, 32, 16, 8))
    nper = N // T
    const2 = lambda r: (0, 0)
    out = pl.pallas_call(
        functools.partial(_edge_body, t=T, k=k, fin=Fin),
        out_shape=jax.ShapeDtypeStruct((B, 2 * Fout, N), f32),
        grid=(R // T,),
        in_specs=[
            pl.BlockSpec((T, C), lambda r: (r, 0)),
            pl.BlockSpec((N, 1, C), lambda r, m=nper: (r // m, 0, 0)),
            pl.BlockSpec((T, k), lambda r: (r, 0), memory_space=pltpu.SMEM),
            pl.BlockSpec((1, Fout, 1), lambda r, m=nper: (r // m, 0, 0)),
            pl.BlockSpec((C, 96), const2),
            pl.BlockSpec((C, 96), const2),
            pl.BlockSpec((16, 64), const2),
            pl.BlockSpec((2, 64), const2),
            pl.BlockSpec((64, Fin), const2),
            pl.BlockSpec((2, Fin), const2),
            pl.BlockSpec((k, Fin, Fout), lambda r: (0, 0, 0)),
            pl.BlockSpec((2, Fout), const2),
            pl.BlockSpec((2, Fout), const2),
        ],
        out_specs=pl.BlockSpec((1, 2 * Fout, T),
                               lambda r, m=nper: (r // m, 0, r % m)),
        scratch_shapes=[pltpu.VMEM((k * T, C), f32)],
        compiler_params=pltpu.CompilerParams(
            dimension_semantics=("parallel",)),
    )(cen2, xpc3, idx2, ys3,
      wn, wcm,
      wa1, sta1, wa2, sta2,
      w2m, st2, stu)
    return out
